# baseline (speedup 1.0000x reference)
import jax
import jax.numpy as jnp
from jax import lax
from jax.experimental import pallas as pl
from jax.experimental.pallas import tpu as pltpu
from jax.experimental.pallas import tpu_sc as plsc

HIDDEN = 64
GAMMA_VAL = 12.0
BATCH_N = 16384
LANES = 16
NENT = 1000000

NUM_CORES = 2
NUM_SUBCORES = 16
NW = NUM_CORES * NUM_SUBCORES
BW = BATCH_N // NW
CH = 128
NCH = BW // CH
CGROUPS = CH // LANES


def _score_body(samp, ent, rel, out,
                samp_v, hi_v, ri_v, ti_v, ho_v, ro_v, to_v,
                h_v, r_v, t_v, out_v, sem):
    wid = lax.axis_index("s") * NUM_CORES + lax.axis_index("c")
    base = wid * BW

    pltpu.sync_copy(samp.at[pl.ds(base * 3, BW * 3)], samp_v)

    iota = lax.iota(jnp.int32, LANES)
    stride3 = iota * 3

    for g in range(BW // LANES):
        flat = g * LANES * 3
        c, o = divmod(g * LANES, CH)
        dst = pl.ds(o, LANES)
        hi = plsc.load_gather(samp_v, [stride3 + flat])
        ri = plsc.load_gather(samp_v, [stride3 + (flat + 1)])
        ti = plsc.load_gather(samp_v, [stride3 + (flat + 2)])
        hi_v[c, dst] = hi >> 1
        ri_v[c, dst] = ri >> 1
        ti_v[c, dst] = ti >> 1
        ho_v[c, dst] = (hi & 1) * HIDDEN
        ro_v[c, dst] = (ri & 1) * HIDDEN
        to_v[c, dst] = (ti & 1) * HIDDEN

    def chunk(c, carry):
        cph = pltpu.async_copy(ent.at[hi_v.at[c]], h_v, sem)
        cpr = pltpu.async_copy(rel.at[ri_v.at[c]], r_v, sem)
        cpt = pltpu.async_copy(ent.at[ti_v.at[c]], t_v, sem)
        cph.wait()
        cpr.wait()
        cpt.wait()

        def body(g, carry):
            rows = g * LANES + iota
            cfull = jnp.full((LANES,), c, jnp.int32)
            cols = g * LANES + iota
            ho = plsc.load_gather(ho_v, [cfull, cols])
            ro = plsc.load_gather(ro_v, [cfull, cols])
            to = plsc.load_gather(to_v, [cfull, cols])
            acc = jnp.zeros((LANES,), jnp.float32)
            for d in range(HIDDEN):
                hv = plsc.load_gather(h_v, [rows, ho + d])
                rv = plsc.load_gather(r_v, [rows, ro + d])
                tv = plsc.load_gather(t_v, [rows, to + d])
                acc = acc + jnp.abs(hv + rv - tv)
            out_v[pl.ds(c * CH + g * LANES, LANES)] = GAMMA_VAL - acc
            return carry

        lax.fori_loop(0, CGROUPS, body, 0)
        return carry

    lax.fori_loop(0, NCH, chunk, 0)
    pltpu.sync_copy(out_v, out.at[pl.ds(base, BW)])


_sc_call = pl.kernel(
    _score_body,
    out_type=jax.ShapeDtypeStruct((BATCH_N,), jnp.float32),
    mesh=plsc.VectorSubcoreMesh(core_axis_name="c", subcore_axis_name="s"),
    scratch_types=[
        pltpu.VMEM((BW * 3,), jnp.int32),
        pltpu.VMEM((NCH, CH), jnp.int32),
        pltpu.VMEM((NCH, CH), jnp.int32),
        pltpu.VMEM((NCH, CH), jnp.int32),
        pltpu.VMEM((NCH, CH), jnp.int32),
        pltpu.VMEM((NCH, CH), jnp.int32),
        pltpu.VMEM((NCH, CH), jnp.int32),
        pltpu.VMEM((CH, 2 * HIDDEN), jnp.float32),
        pltpu.VMEM((CH, 2 * HIDDEN), jnp.float32),
        pltpu.VMEM((CH, 2 * HIDDEN), jnp.float32),
        pltpu.VMEM((BW,), jnp.float32),
        pltpu.SemaphoreType.DMA,
    ],
    compiler_params=pltpu.CompilerParams(needs_layout_passes=False),
)


@jax.jit
def kernel(sample, entity_embedding, relation_embedding):
    samp = sample.reshape(BATCH_N * 3)
    ent2 = entity_embedding.reshape(NENT // 2, 2 * HIDDEN)
    rel2 = relation_embedding.reshape(NENT // 2, 2 * HIDDEN)
    score = _sc_call(samp, ent2, rel2)
    return score.reshape(BATCH_N, 1)


# speedup vs baseline: 1.0065x; 1.0065x over previous
import jax
import jax.numpy as jnp
from jax import lax
from jax.experimental import pallas as pl
from jax.experimental.pallas import tpu as pltpu
from jax.experimental.pallas import tpu_sc as plsc

HIDDEN = 64
GAMMA_VAL = 12.0
BATCH_N = 16384
LANES = 16
PACK = 128

NUM_CORES = 2
NUM_SUBCORES = 16
NW = NUM_CORES * NUM_SUBCORES
BW = BATCH_N // NW
CH = 128
NCH = BW // CH
CGROUPS = CH // LANES


def _score_body(samp, ent, rel, out, samp_v, hidx_v, ridx_v, tidx_v,
                hcol_v, rcol_v, tcol_v, h_v, r_v, t_v, out_v, sem):
    wid = lax.axis_index("s") * NUM_CORES + lax.axis_index("c")
    base = wid * BW

    pltpu.sync_copy(samp.at[pl.ds(base * 3, BW * 3)], samp_v)

    iota = lax.iota(jnp.int32, LANES)

    def chunk(c, carry):
        def deint(g, carry):
            rows3 = (c * CH + g * LANES + iota) * 3
            dst = pl.ds(g * LANES, LANES)
            hv = plsc.load_gather(samp_v, [rows3])
            rv = plsc.load_gather(samp_v, [rows3 + 1])
            tv = plsc.load_gather(samp_v, [rows3 + 2])
            hidx_v[dst] = jnp.right_shift(hv, 1)
            ridx_v[dst] = jnp.right_shift(rv, 1)
            tidx_v[dst] = jnp.right_shift(tv, 1)
            hcol_v[dst] = jnp.bitwise_and(hv, 1) * HIDDEN
            rcol_v[dst] = jnp.bitwise_and(rv, 1) * HIDDEN
            tcol_v[dst] = jnp.bitwise_and(tv, 1) * HIDDEN
            return carry

        lax.fori_loop(0, CGROUPS, deint, 0)

        cph = pltpu.async_copy(ent.at[hidx_v], h_v, sem)
        cpr = pltpu.async_copy(rel.at[ridx_v], r_v, sem)
        cpt = pltpu.async_copy(ent.at[tidx_v], t_v, sem)
        cph.wait()
        cpr.wait()
        cpt.wait()

        def body(g, carry):
            rows = g * LANES + iota
            hcol = plsc.load_gather(hcol_v, [rows])
            rcol = plsc.load_gather(rcol_v, [rows])
            tcol = plsc.load_gather(tcol_v, [rows])
            acc = jnp.zeros((LANES,), jnp.float32)
            for d in range(HIDDEN):
                hv = plsc.load_gather(h_v, [rows, hcol + d])
                rv = plsc.load_gather(r_v, [rows, rcol + d])
                tv = plsc.load_gather(t_v, [rows, tcol + d])
                acc = acc + jnp.abs(hv + rv - tv)
            out_v[pl.ds(c * CH + g * LANES, LANES)] = GAMMA_VAL - acc
            return carry

        lax.fori_loop(0, CGROUPS, body, 0)
        return carry

    lax.fori_loop(0, NCH, chunk, 0)
    pltpu.sync_copy(out_v, out.at[pl.ds(base, BW)])


_sc_call = pl.kernel(
    _score_body,
    out_type=jax.ShapeDtypeStruct((BATCH_N,), jnp.float32),
    mesh=plsc.VectorSubcoreMesh(core_axis_name="c", subcore_axis_name="s"),
    scratch_types=[
        pltpu.VMEM((BW * 3,), jnp.int32),
        pltpu.VMEM((CH,), jnp.int32),
        pltpu.VMEM((CH,), jnp.int32),
        pltpu.VMEM((CH,), jnp.int32),
        pltpu.VMEM((CH,), jnp.int32),
        pltpu.VMEM((CH,), jnp.int32),
        pltpu.VMEM((CH,), jnp.int32),
        pltpu.VMEM((CH, PACK), jnp.float32),
        pltpu.VMEM((CH, PACK), jnp.float32),
        pltpu.VMEM((CH, PACK), jnp.float32),
        pltpu.VMEM((BW,), jnp.float32),
        pltpu.SemaphoreType.DMA,
    ],
    compiler_params=pltpu.CompilerParams(needs_layout_passes=False),
)


@jax.jit
def kernel(sample, entity_embedding, relation_embedding):
    samp = sample.reshape(BATCH_N * 3)
    ent2 = entity_embedding.reshape(-1, PACK)
    rel2 = relation_embedding.reshape(-1, PACK)
    score = _sc_call(samp, ent2, rel2)
    return score.reshape(BATCH_N, 1)


# speedup vs baseline: 1.0344x; 1.0277x over previous
import jax
import jax.numpy as jnp
from jax import lax
from jax.experimental import pallas as pl
from jax.experimental.pallas import tpu as pltpu
from jax.experimental.pallas import tpu_sc as plsc

HIDDEN = 64
GAMMA_VAL = 12.0
BATCH_N = 16384
LANES = 16
PACK = 128

NUM_CORES = 2
NUM_SUBCORES = 16
NW = NUM_CORES * NUM_SUBCORES
BW = BATCH_N // NW
CH = 128
NCH = BW // CH
CGROUPS = CH // LANES


def _score_body(samp, ent, rel, out, samp_v, hidx_v, ridx_v, tidx_v,
                hcol_v, rcol_v, tcol_v, h_v, r_v, t_v, part_v, out_v, sem):
    wid = lax.axis_index("s") * NUM_CORES + lax.axis_index("c")
    base = wid * BW

    pltpu.sync_copy(samp.at[pl.ds(base * 3, BW * 3)], samp_v)

    iota = lax.iota(jnp.int32, LANES)

    def chunk(c, carry):
        def deint(g, carry):
            rows3 = (c * CH + g * LANES + iota) * 3
            dst = pl.ds(g * LANES, LANES)
            hv = plsc.load_gather(samp_v, [rows3])
            rv = plsc.load_gather(samp_v, [rows3 + 1])
            tv = plsc.load_gather(samp_v, [rows3 + 2])
            hidx_v[dst] = jnp.right_shift(hv, 1)
            ridx_v[dst] = jnp.right_shift(rv, 1)
            tidx_v[dst] = jnp.right_shift(tv, 1)
            hcol_v[dst] = jnp.bitwise_and(hv, 1) * HIDDEN
            rcol_v[dst] = jnp.bitwise_and(rv, 1) * HIDDEN
            tcol_v[dst] = jnp.bitwise_and(tv, 1) * HIDDEN
            return carry

        lax.fori_loop(0, CGROUPS, deint, 0)

        cph = pltpu.async_copy(ent.at[hidx_v], h_v, sem)
        cpr = pltpu.async_copy(rel.at[ridx_v], r_v, sem)
        cpt = pltpu.async_copy(ent.at[tidx_v], t_v, sem)
        cph.wait()
        cpr.wait()
        cpt.wait()

        def sample_body(s, carry):
            svec = jnp.full((LANES,), s, jnp.int32)
            hmask = plsc.load_gather(hcol_v, [svec]) != 0
            rmask = plsc.load_gather(rcol_v, [svec]) != 0
            tmask = plsc.load_gather(tcol_v, [svec]) != 0
            acc = jnp.zeros((LANES,), jnp.float32)
            for k in range(HIDDEN // LANES):
                lo = pl.ds(k * LANES, LANES)
                hi = pl.ds(HIDDEN + k * LANES, LANES)
                hv = jnp.where(hmask, h_v[s, hi], h_v[s, lo])
                rv = jnp.where(rmask, r_v[s, hi], r_v[s, lo])
                tv = jnp.where(tmask, t_v[s, hi], t_v[s, lo])
                acc = acc + jnp.abs(hv + rv - tv)
            part_v[s, :] = acc
            return carry

        lax.fori_loop(0, CH, sample_body, 0)

        def red(g, carry):
            rows = g * LANES + iota
            acc16 = jnp.zeros((LANES,), jnp.float32)
            for k in range(LANES):
                acc16 = acc16 + plsc.load_gather(
                    part_v, [rows, jnp.full((LANES,), k, jnp.int32)])
            out_v[pl.ds(c * CH + g * LANES, LANES)] = GAMMA_VAL - acc16
            return carry

        lax.fori_loop(0, CGROUPS, red, 0)
        return carry

    lax.fori_loop(0, NCH, chunk, 0)
    pltpu.sync_copy(out_v, out.at[pl.ds(base, BW)])


_sc_call = pl.kernel(
    _score_body,
    out_type=jax.ShapeDtypeStruct((BATCH_N,), jnp.float32),
    mesh=plsc.VectorSubcoreMesh(core_axis_name="c", subcore_axis_name="s"),
    scratch_types=[
        pltpu.VMEM((BW * 3,), jnp.int32),
        pltpu.VMEM((CH,), jnp.int32),
        pltpu.VMEM((CH,), jnp.int32),
        pltpu.VMEM((CH,), jnp.int32),
        pltpu.VMEM((CH,), jnp.int32),
        pltpu.VMEM((CH,), jnp.int32),
        pltpu.VMEM((CH,), jnp.int32),
        pltpu.VMEM((CH, PACK), jnp.float32),
        pltpu.VMEM((CH, PACK), jnp.float32),
        pltpu.VMEM((CH, PACK), jnp.float32),
        pltpu.VMEM((CH, LANES), jnp.float32),
        pltpu.VMEM((BW,), jnp.float32),
        pltpu.SemaphoreType.DMA,
    ],
    compiler_params=pltpu.CompilerParams(needs_layout_passes=False),
)


@jax.jit
def kernel(sample, entity_embedding, relation_embedding):
    samp = sample.reshape(BATCH_N * 3)
    ent2 = entity_embedding.reshape(-1, PACK)
    rel2 = relation_embedding.reshape(-1, PACK)
    score = _sc_call(samp, ent2, rel2)
    return score.reshape(BATCH_N, 1)
